# baseline (speedup 1.0000x reference)
import functools

import jax
import jax.numpy as jnp
from jax.experimental import pallas as pl

B, N, K, NUM_CLASSES = 16, 1024, 20, 42
NPOINTS = [1024, 1024, 1024, 1024, 256, 256, 64, 64]


def _knn_idx(pos, k):
    sq = jnp.sum(pos * pos, axis=-1)
    d = sq[:, :, None] - 2.0 * jnp.einsum("bnd,bmd->bnm", pos, pos) + sq[:, None, :]
    _, idx = jax.lax.top_k(-d, k)
    return idx


def _gather_nb(feat, idx):
    return jax.vmap(lambda f, i: f[i])(feat, idx)


def _lpfa_initial(pos, w):
    idx = _knn_idx(pos, K)
    nb = _gather_nb(pos, idx)
    ctr = jnp.broadcast_to(pos[:, :, None, :], nb.shape)
    feat = jnp.concatenate([ctr, nb, nb - ctr], axis=-1)
    out = jax.nn.relu(jnp.einsum("bnkc,cd->bnkd", feat, w))
    return jnp.max(out, axis=2)


def _cic(pos, feat, npoint, w1, we, w2, ws):
    n = pos.shape[1]
    if npoint < n:
        stride = n // npoint
        pos = pos[:, ::stride][:, :npoint]
        feat = feat[:, ::stride][:, :npoint]
    sc = feat if ws is None else feat @ ws
    x = jax.nn.relu(feat @ w1)
    idx = _knn_idx(pos, K)
    nb = _gather_nb(x, idx)
    edge = nb - x[:, :, None, :]
    e = jax.nn.relu(jnp.einsum("bnkc,cd->bnkd", edge, we))
    agg = jnp.max(e, axis=2) + x
    out = agg @ w2
    return pos, jax.nn.relu(out + sc)


def _head_kernel(x_ref, w_ref, b_ref, o_ref):
    x = x_ref[...]
    logits = x @ w_ref[...] + b_ref[...][None, :]
    m = jnp.max(logits, axis=-1, keepdims=True)
    s = logits - m
    lse = jnp.log(jnp.sum(jnp.exp(s), axis=-1, keepdims=True))
    o_ref[...] = s - lse


def kernel(xyz, params):
    pos = jnp.transpose(xyz[:, :3, :], (0, 2, 1))
    feat = _lpfa_initial(pos, params["lpfa_w"])
    p = pos
    for i in range(8):
        ws = params.get(f"cic{i}_ws")
        p, feat = _cic(p, feat, NPOINTS[i], params[f"cic{i}_w1"],
                       params[f"cic{i}_we"], params[f"cic{i}_w2"], ws)
    x = jax.nn.relu(feat @ params["conv0_w"])
    x_max = jnp.max(x, axis=1)
    x_avg = jnp.mean(x, axis=1)
    x = jnp.concatenate([x_max, x_avg], axis=-1)
    x = jax.nn.relu(x @ params["fc1_w"])
    logits = pl.pallas_call(
        _head_kernel,
        out_shape=jax.ShapeDtypeStruct((B, NUM_CLASSES), jnp.float32),
    )(x, params["fc2_w"], params["fc2_b"])
    return logits, jnp.transpose(feat, (0, 2, 1))


# speedup vs baseline: 4.6795x; 4.6795x over previous
import functools

import jax
import jax.numpy as jnp
from jax.experimental import pallas as pl

B, N, K, NUM_CLASSES = 16, 1024, 20, 42
NPOINTS = [1024, 1024, 1024, 1024, 256, 256, 64, 64]
CFG = [(32, 64, 2), (64, 64, 4), (64, 128, 2), (128, 128, 4),
       (128, 256, 2), (256, 256, 4), (256, 512, 2), (512, 512, 4)]
KPAD = 32

_BIG = 3.0e38
_DN = (((1,), (0,)), ((), ()))


def _mmbf(a, b):
    return jax.lax.dot_general(a.astype(jnp.bfloat16), b.astype(jnp.bfloat16),
                               _DN, preferred_element_type=jnp.float32)


def _gather_rows(onehot_t, v):
    return jax.lax.dot_general(onehot_t, v, (((0,), (0,)), ((), ())),
                               preferred_element_type=jnp.float32,
                               precision=jax.lax.Precision.HIGHEST)


def _knn_kernel(pos_ref, post_ref, idx_ref, *, n, k):
    pos = pos_ref[0]
    post = post_ref[0]
    sq_col = jnp.sum(pos * pos, axis=1, keepdims=True)
    sq_row = jnp.sum(post * post, axis=0, keepdims=True)
    dot = _mmbf(pos, post)
    d = sq_col - 2.0 * dot + sq_row
    irow = jax.lax.broadcasted_iota(jnp.int32, (n, n), 0)
    krow = jax.lax.broadcasted_iota(jnp.int32, (KPAD, n), 0)
    idxmat = jnp.zeros((KPAD, n), jnp.int32)
    for kk in range(k):
        m = jnp.min(d, axis=0, keepdims=True)
        cand = jnp.where(d == m, irow, n)
        j = jnp.min(cand, axis=0, keepdims=True)
        idxmat = jnp.where(krow == kk, j, idxmat)
        d = jnp.where(irow == j, _BIG, d)
    idx_ref[0] = idxmat


def _knn(pos):
    n = pos.shape[1]
    pos8 = jnp.pad(pos, ((0, 0), (0, 0), (0, 5)))
    post8 = jnp.transpose(pos8, (0, 2, 1))
    return pl.pallas_call(
        functools.partial(_knn_kernel, n=n, k=K),
        grid=(B,),
        in_specs=[
            pl.BlockSpec((1, n, 8), lambda b: (b, 0, 0)),
            pl.BlockSpec((1, 8, n), lambda b: (b, 0, 0)),
        ],
        out_specs=pl.BlockSpec((1, KPAD, n), lambda b: (b, 0, 0)),
        out_shape=jax.ShapeDtypeStruct((B, KPAD, n), jnp.int32),
    )(pos8, post8)


def _extract_row(krow, idxmat, kk):
    return jnp.max(jnp.where(krow == kk, idxmat, 0), axis=0, keepdims=True)


def _lpfa_kernel(pos_ref, idx_ref, w24_ref, out_ref, *, n, k):
    pos = pos_ref[0]
    w24 = w24_ref[...]
    irow = jax.lax.broadcasted_iota(jnp.int32, (n, n), 0)
    krow = jax.lax.broadcasted_iota(jnp.int32, (KPAD, n), 0)
    idxmat = idx_ref[0]

    def body(kk, acc):
        idxk = _extract_row(krow, idxmat, kk)
        onehot_t = (irow == idxk).astype(jnp.float32)
        nb = _gather_rows(onehot_t, pos)
        feat9 = jnp.concatenate([pos, nb, nb - pos], axis=1)
        e = jnp.maximum(_mmbf(feat9, w24), 0.0)
        return jnp.maximum(acc, e)

    out_ref[0] = jax.lax.fori_loop(
        0, k, body, jnp.zeros((n, 32), jnp.float32))


def _lpfa(pos, idx, w):
    n = pos.shape[1]
    z5 = jnp.zeros((5, 32), jnp.float32)
    w24 = jnp.concatenate([w[0:3], z5, w[3:6], z5, w[6:9], z5], axis=0)
    pos8 = jnp.pad(pos, ((0, 0), (0, 0), (0, 5)))
    return pl.pallas_call(
        functools.partial(_lpfa_kernel, n=n, k=K),
        grid=(B,),
        in_specs=[
            pl.BlockSpec((1, n, 8), lambda b: (b, 0, 0)),
            pl.BlockSpec((1, KPAD, n), lambda b: (b, 0, 0)),
            pl.BlockSpec((24, 32), lambda b: (0, 0)),
        ],
        out_specs=pl.BlockSpec((1, n, 32), lambda b: (b, 0, 0)),
        out_shape=jax.ShapeDtypeStruct((B, n, 32), jnp.float32),
    )(pos8, idx, w24)


def _cic_kernel(feat_ref, idx_ref, w1_ref, we_ref, w2_ref, ws_ref, out_ref,
                *, n, k, has_ws):
    feat = feat_ref[0]
    x = jnp.maximum(_mmbf(feat, w1_ref[...]), 0.0)
    we = we_ref[...]
    irow = jax.lax.broadcasted_iota(jnp.int32, (n, n), 0)
    krow = jax.lax.broadcasted_iota(jnp.int32, (KPAD, n), 0)
    idxmat = idx_ref[0]

    def body(kk, acc):
        idxk = _extract_row(krow, idxmat, kk)
        onehot_t = (irow == idxk).astype(jnp.float32)
        nb = _gather_rows(onehot_t, x)
        e = jnp.maximum(_mmbf(nb - x, we), 0.0)
        return jnp.maximum(acc, e)

    acc = jax.lax.fori_loop(0, k, body, jnp.zeros_like(x))
    agg = acc + x
    out = _mmbf(agg, w2_ref[...])
    if has_ws:
        sc = _mmbf(feat, ws_ref[...])
    else:
        sc = feat
    out_ref[0] = jnp.maximum(out + sc, 0.0)


def _cic(feat, idx, w1, we, w2, ws):
    n = feat.shape[1]
    cin, mid = w1.shape
    cout = w2.shape[1]
    has_ws = ws is not None
    ws_in = ws if has_ws else jnp.zeros((cin, cout), jnp.float32)
    return pl.pallas_call(
        functools.partial(_cic_kernel, n=n, k=K, has_ws=has_ws),
        grid=(B,),
        in_specs=[
            pl.BlockSpec((1, n, cin), lambda b: (b, 0, 0)),
            pl.BlockSpec((1, KPAD, n), lambda b: (b, 0, 0)),
            pl.BlockSpec((cin, mid), lambda b: (0, 0)),
            pl.BlockSpec((mid, mid), lambda b: (0, 0)),
            pl.BlockSpec((mid, cout), lambda b: (0, 0)),
            pl.BlockSpec((cin, cout), lambda b: (0, 0)),
        ],
        out_specs=pl.BlockSpec((1, n, cout), lambda b: (b, 0, 0)),
        out_shape=jax.ShapeDtypeStruct((B, n, cout), jnp.float32),
    )(feat, idx, w1, we, w2, ws_in)


def _head_kernel(feat_ref, conv0_ref, fc1_ref, fc2_ref, fc2b_ref, out_ref,
                 *, npts):
    feat = feat_ref[...].reshape(B * npts, 512)
    x = jnp.maximum(_mmbf(feat, conv0_ref[...]), 0.0)
    x3 = x.reshape(B, npts, N)
    x_max = jnp.max(x3, axis=1)
    x_avg = jnp.mean(x3, axis=1)
    h1 = jnp.maximum(_mmbf(x_max, fc1_ref[0:N, :])
                     + _mmbf(x_avg, fc1_ref[N:2 * N, :]), 0.0)
    logits = _mmbf(h1, fc2_ref[...]) + fc2b_ref[...]
    m = jnp.max(logits, axis=1, keepdims=True)
    s = logits - m
    lse = jnp.log(jnp.sum(jnp.exp(s), axis=1, keepdims=True))
    out_ref[...] = s - lse


def _head(feat, conv0_w, fc1_w, fc2_w, fc2_b):
    npts = feat.shape[1]
    return pl.pallas_call(
        functools.partial(_head_kernel, npts=npts),
        out_shape=jax.ShapeDtypeStruct((B, NUM_CLASSES), jnp.float32),
    )(feat, conv0_w, fc1_w, fc2_w, fc2_b.reshape(1, NUM_CLASSES))


def kernel(xyz, params):
    pos = jnp.transpose(xyz[:, :3, :], (0, 2, 1))
    idx = _knn(pos)
    feat = _lpfa(pos, idx, params["lpfa_w"])
    p = pos
    for i in range(8):
        npoint = NPOINTS[i]
        n = p.shape[1]
        if npoint < n:
            stride = n // npoint
            p = p[:, ::stride][:, :npoint]
            feat = feat[:, ::stride][:, :npoint]
            idx = _knn(p)
        feat = _cic(feat, idx, params[f"cic{i}_w1"], params[f"cic{i}_we"],
                    params[f"cic{i}_w2"], params.get(f"cic{i}_ws"))
    logits = _head(feat, params["conv0_w"], params["fc1_w"],
                   params["fc2_w"], params["fc2_b"])
    return logits, jnp.transpose(feat, (0, 2, 1))


# speedup vs baseline: 13.1047x; 2.8004x over previous
import functools

import jax
import jax.numpy as jnp
from jax.experimental import pallas as pl

B, N, K, NUM_CLASSES = 16, 1024, 20, 42
NPOINTS = [1024, 1024, 1024, 1024, 256, 256, 64, 64]
CFG = [(32, 64, 2), (64, 64, 4), (64, 128, 2), (128, 128, 4),
       (128, 256, 2), (256, 256, 4), (256, 512, 2), (512, 512, 4)]
KPAD = 32

_BIG = 3.0e38
_DN = (((1,), (0,)), ((), ()))


def _mmbf(a, b):
    return jax.lax.dot_general(a.astype(jnp.bfloat16), b.astype(jnp.bfloat16),
                               _DN, preferred_element_type=jnp.float32)


def _split3(v):
    v1 = v.astype(jnp.bfloat16)
    r1 = v - v1.astype(jnp.float32)
    v2 = r1.astype(jnp.bfloat16)
    v3 = (r1 - v2.astype(jnp.float32)).astype(jnp.bfloat16)
    return jnp.concatenate([v1, v2, v3], axis=1)


def _gather3(onehot_t, vcat, c):
    g = jax.lax.dot_general(onehot_t, vcat, (((0,), (0,)), ((), ())),
                            preferred_element_type=jnp.float32)
    return g[:, 0:c] + g[:, c:2 * c] + g[:, 2 * c:3 * c]


def _knn_kernel(pos_ref, post_ref, idx_ref, *, n, k):
    pos = pos_ref[0]
    post = post_ref[0]
    sq_col = jnp.sum(pos * pos, axis=1, keepdims=True)
    sq_row = jnp.sum(post * post, axis=0, keepdims=True)
    dot = _mmbf(pos, post)
    d = sq_col - 2.0 * dot + sq_row
    irow = jax.lax.broadcasted_iota(jnp.int32, (n, n), 0)
    krow = jax.lax.broadcasted_iota(jnp.int32, (KPAD, n), 0)
    idxmat = jnp.zeros((KPAD, n), jnp.int32)
    for kk in range(k):
        m = jnp.min(d, axis=0, keepdims=True)
        cand = jnp.where(d == m, irow, n)
        j = jnp.min(cand, axis=0, keepdims=True)
        idxmat = jnp.where(krow == kk, j, idxmat)
        d = jnp.where(irow == j, _BIG, d)
    idx_ref[0] = idxmat


def _knn(pos):
    n = pos.shape[1]
    pos8 = jnp.pad(pos, ((0, 0), (0, 0), (0, 5)))
    post8 = jnp.transpose(pos8, (0, 2, 1))
    return pl.pallas_call(
        functools.partial(_knn_kernel, n=n, k=K),
        grid=(B,),
        in_specs=[
            pl.BlockSpec((1, n, 8), lambda b: (b, 0, 0)),
            pl.BlockSpec((1, 8, n), lambda b: (b, 0, 0)),
        ],
        out_specs=pl.BlockSpec((1, KPAD, n), lambda b: (b, 0, 0)),
        out_shape=jax.ShapeDtypeStruct((B, KPAD, n), jnp.int32),
    )(pos8, post8)


def _extract_row(krow, idxmat, kk):
    return jnp.max(jnp.where(krow == kk, idxmat, 0), axis=0, keepdims=True)


def _lpfa_kernel(pos_ref, idx_ref, w24_ref, out_ref, *, n, k):
    pos = pos_ref[0]
    w24 = w24_ref[...]
    poscat = _split3(pos)
    irow = jax.lax.broadcasted_iota(jnp.int32, (n, n), 0)
    krow = jax.lax.broadcasted_iota(jnp.int32, (KPAD, n), 0)
    idxmat = idx_ref[0]

    def body(kk, acc):
        idxk = _extract_row(krow, idxmat, kk)
        onehot_t = (irow == idxk).astype(jnp.bfloat16)
        nb = _gather3(onehot_t, poscat, 8)
        feat9 = jnp.concatenate([pos, nb, nb - pos], axis=1)
        e = jnp.maximum(_mmbf(feat9, w24), 0.0)
        return jnp.maximum(acc, e)

    out_ref[0] = jax.lax.fori_loop(
        0, k, body, jnp.zeros((n, 32), jnp.float32))


def _lpfa(pos, idx, w):
    n = pos.shape[1]
    z5 = jnp.zeros((5, 32), jnp.float32)
    w24 = jnp.concatenate([w[0:3], z5, w[3:6], z5, w[6:9], z5], axis=0)
    pos8 = jnp.pad(pos, ((0, 0), (0, 0), (0, 5)))
    return pl.pallas_call(
        functools.partial(_lpfa_kernel, n=n, k=K),
        grid=(B,),
        in_specs=[
            pl.BlockSpec((1, n, 8), lambda b: (b, 0, 0)),
            pl.BlockSpec((1, KPAD, n), lambda b: (b, 0, 0)),
            pl.BlockSpec((24, 32), lambda b: (0, 0)),
        ],
        out_specs=pl.BlockSpec((1, n, 32), lambda b: (b, 0, 0)),
        out_shape=jax.ShapeDtypeStruct((B, n, 32), jnp.float32),
    )(pos8, idx, w24)


def _cic_kernel(feat_ref, idx_ref, w1_ref, we_ref, w2_ref, ws_ref, out_ref,
                *, n, k, has_ws):
    feat = feat_ref[0]
    x = jnp.maximum(_mmbf(feat, w1_ref[...]), 0.0)
    mid = x.shape[1]
    xcat = _split3(x)
    we = we_ref[...]
    irow = jax.lax.broadcasted_iota(jnp.int32, (n, n), 0)
    krow = jax.lax.broadcasted_iota(jnp.int32, (KPAD, n), 0)
    idxmat = idx_ref[0]

    def body(kk, acc):
        idxk = _extract_row(krow, idxmat, kk)
        onehot_t = (irow == idxk).astype(jnp.bfloat16)
        nb = _gather3(onehot_t, xcat, mid)
        e = jnp.maximum(_mmbf(nb - x, we), 0.0)
        return jnp.maximum(acc, e)

    acc = jax.lax.fori_loop(0, k, body, jnp.zeros_like(x))
    agg = acc + x
    out = _mmbf(agg, w2_ref[...])
    if has_ws:
        sc = _mmbf(feat, ws_ref[...])
    else:
        sc = feat
    out_ref[0] = jnp.maximum(out + sc, 0.0)


def _cic(feat, idx, w1, we, w2, ws):
    n = feat.shape[1]
    cin, mid = w1.shape
    cout = w2.shape[1]
    has_ws = ws is not None
    ws_in = ws if has_ws else jnp.zeros((cin, cout), jnp.float32)
    return pl.pallas_call(
        functools.partial(_cic_kernel, n=n, k=K, has_ws=has_ws),
        grid=(B,),
        in_specs=[
            pl.BlockSpec((1, n, cin), lambda b: (b, 0, 0)),
            pl.BlockSpec((1, KPAD, n), lambda b: (b, 0, 0)),
            pl.BlockSpec((cin, mid), lambda b: (0, 0)),
            pl.BlockSpec((mid, mid), lambda b: (0, 0)),
            pl.BlockSpec((mid, cout), lambda b: (0, 0)),
            pl.BlockSpec((cin, cout), lambda b: (0, 0)),
        ],
        out_specs=pl.BlockSpec((1, n, cout), lambda b: (b, 0, 0)),
        out_shape=jax.ShapeDtypeStruct((B, n, cout), jnp.float32),
    )(feat, idx, w1, we, w2, ws_in)


def _head_kernel(feat_ref, conv0_ref, fc1_ref, fc2_ref, fc2b_ref, out_ref,
                 *, npts):
    feat = feat_ref[...].reshape(B * npts, 512)
    x = jnp.maximum(_mmbf(feat, conv0_ref[...]), 0.0)
    x3 = x.reshape(B, npts, N)
    x_max = jnp.max(x3, axis=1)
    x_avg = jnp.mean(x3, axis=1)
    h1 = jnp.maximum(_mmbf(x_max, fc1_ref[0:N, :])
                     + _mmbf(x_avg, fc1_ref[N:2 * N, :]), 0.0)
    logits = _mmbf(h1, fc2_ref[...]) + fc2b_ref[...]
    m = jnp.max(logits, axis=1, keepdims=True)
    s = logits - m
    lse = jnp.log(jnp.sum(jnp.exp(s), axis=1, keepdims=True))
    out_ref[...] = s - lse


def _head(feat, conv0_w, fc1_w, fc2_w, fc2_b):
    npts = feat.shape[1]
    return pl.pallas_call(
        functools.partial(_head_kernel, npts=npts),
        out_shape=jax.ShapeDtypeStruct((B, NUM_CLASSES), jnp.float32),
    )(feat, conv0_w, fc1_w, fc2_w, fc2_b.reshape(1, NUM_CLASSES))


def kernel(xyz, params):
    pos = jnp.transpose(xyz[:, :3, :], (0, 2, 1))
    idx = _knn(pos)
    feat = _lpfa(pos, idx, params["lpfa_w"])
    p = pos
    for i in range(8):
        npoint = NPOINTS[i]
        n = p.shape[1]
        if npoint < n:
            stride = n // npoint
            p = p[:, ::stride][:, :npoint]
            feat = feat[:, ::stride][:, :npoint]
            idx = _knn(p)
        feat = _cic(feat, idx, params[f"cic{i}_w1"], params[f"cic{i}_we"],
                    params[f"cic{i}_w2"], params.get(f"cic{i}_ws"))
    logits = _head(feat, params["conv0_w"], params["fc1_w"],
                   params["fc2_w"], params["fc2_b"])
    return logits, jnp.transpose(feat, (0, 2, 1))
